# baseline (speedup 1.0000x reference)
import jax
import jax.numpy as jnp
import numpy as np
from jax.experimental import pallas as pl
from jax.experimental.pallas import tpu as pltpu

_N = 1024
_C = 8
_GAMMA = 1.0
_PER = 0.001
_SIG1 = float(1.0 / (1.0 + np.exp(-1.0)))
_CHUNK = 64
_NCHUNK = _N // _CHUNK


def _loss_body(preds_ref, lab_ref, maskf_ref, a_ref, diag_ref, out_ref,
               t_ref, wt_ref):
    preds = preds_ref[...]
    labels = lab_ref[...]
    maskf = maskf_ref[...]
    diag_col = diag_ref[...]

    mx = jnp.max(preds, axis=1, keepdims=True)
    lse = jnp.log(jnp.sum(jnp.exp(preds - mx), axis=1, keepdims=True)) + mx
    logp = preds - lse
    cls_iota = jax.lax.broadcasted_iota(jnp.int32, (_N, _C), 1)
    lab_oh = (cls_iota == labels).astype(jnp.float32)
    ce = -jnp.sum(logp * lab_oh) * (1.0 / _N)

    m_cls = lab_oh * maskf
    ncnt = jnp.sum(m_cls, axis=0, keepdims=True)
    m_bf = m_cls.astype(jnp.bfloat16)

    eg = jnp.exp(_GAMMA * preds)
    u_bf = (m_cls / eg).astype(jnp.bfloat16)

    a_bf = a_ref[...].astype(jnp.bfloat16)
    ones_row = jnp.ones((1, _N), dtype=jnp.bfloat16)
    deg_row = jax.lax.dot_general(ones_row, a_bf, (((1,), (0,)), ((), ())),
                                  preferred_element_type=jnp.float32)
    base_row = 1.0 + _SIG1 * deg_row

    def chunk_body(i, _):
        r0 = i * _CHUNK
        a_chunk_b = a_ref[pl.ds(r0, _CHUNK), :]
        a_chunk = a_chunk_b.astype(jnp.bfloat16)
        inter = jax.lax.dot_general(a_chunk, a_bf, (((1,), (0,)), ((), ())),
                                    preferred_element_type=jnp.float32)
        coef = _SIG1 * (1.0 - diag_ref[pl.ds(r0, _CHUNK), :])
        num = base_row - _SIG1 * inter - jnp.where(a_chunk_b, coef, 0.0)
        rden = pl.reciprocal(1.0 + _SIG1 * inter, approx=True,
                             full_range=False)
        v = pl.reciprocal(1.0 + jnp.exp(num * rden), approx=True,
                          full_range=False)
        v_bf = v.astype(jnp.bfloat16)
        t_chunk = jax.lax.dot_general(v_bf, u_bf, (((1,), (0,)), ((), ())),
                                      preferred_element_type=jnp.float32)
        wt_chunk = jax.lax.dot_general(a_chunk, m_bf, (((1,), (0,)), ((), ())),
                                       preferred_element_type=jnp.float32)
        t_ref[pl.ds(r0, _CHUNK), :] = t_chunk
        wt_ref[pl.ds(r0, _CHUNK), :] = wt_chunk
        return 0

    jax.lax.fori_loop(0, _NCHUNK, chunk_body, 0)

    t = t_ref[...]
    wt = wt_ref[...]
    p_t = jax.lax.dot_general(m_cls, t * eg, (((0,), (0,)), ((), ())),
                              preferred_element_type=jnp.float32)

    zt = ncnt - wt - m_cls * (1.0 - diag_col)
    g_inter = jax.lax.dot_general(wt, wt, (((0,), (0,)), ((), ())),
                                  preferred_element_type=jnp.float32)
    g_sub_t = jax.lax.dot_general(zt, wt, (((0,), (0,)), ((), ())),
                                  preferred_element_type=jnp.float32)

    denom = jnp.reshape(ncnt, (_C, 1)) * ncnt
    recip = jnp.where(denom > 0.0, 1.0 / jnp.where(denom > 0.0, denom, 1.0), 0.0)
    ii = jax.lax.broadcasted_iota(jnp.int32, (_C, _C), 0)
    jj = jax.lax.broadcasted_iota(jnp.int32, (_C, _C), 1)
    keep = jnp.logical_and(jnp.logical_and(g_sub_t > 0.0, g_inter > 0.0),
                           ii != jj)
    pair_loss = jnp.sum(jnp.where(keep, p_t * recip, 0.0))

    out_ref[...] = jnp.reshape(ce + _PER * pair_loss, (1, 1))


def kernel(preds, labels, mask, w_values_dict, adj_matrix):
    del w_values_dict
    adj_b = adj_matrix.astype(bool)
    diag_col = jnp.diagonal(adj_b).astype(jnp.float32).reshape(_N, 1)
    lab = labels.astype(jnp.int32).reshape(_N, 1)
    maskf = mask.astype(jnp.float32).reshape(_N, 1)
    out = pl.pallas_call(
        _loss_body,
        out_shape=jax.ShapeDtypeStruct((1, 1), jnp.float32),
        scratch_shapes=[
            pltpu.VMEM((_N, _C), jnp.float32),
            pltpu.VMEM((_N, _C), jnp.float32),
        ],
    )(preds.astype(jnp.float32), lab, maskf, adj_b, diag_col)
    return out[0, 0]


# speedup vs baseline: 1.0557x; 1.0557x over previous
import jax
import jax.numpy as jnp
import numpy as np
from jax.experimental import pallas as pl

_N = 1024
_C = 8
_GAMMA = 1.0
_PER = 0.001
_SIG1 = float(1.0 / (1.0 + np.exp(-1.0)))
_CHUNK = 64
_NCHUNK = _N // _CHUNK


def _loss_body(preds_ref, lab_ref, maskf_ref, a_ref, aself_ref, diag_ref,
               out_ref):
    preds = preds_ref[...]
    labels = lab_ref[...]
    maskf = maskf_ref[...]
    diag_col = diag_ref[...]

    mx = jnp.max(preds, axis=1, keepdims=True)
    lse = jnp.log(jnp.sum(jnp.exp(preds - mx), axis=1, keepdims=True)) + mx
    logp = preds - lse
    cls_iota = jax.lax.broadcasted_iota(jnp.int32, (_N, _C), 1)
    lab_oh = (cls_iota == labels).astype(jnp.float32)
    ce = -jnp.sum(logp * lab_oh) * (1.0 / _N)

    m_cls = lab_oh * maskf
    ncnt = jnp.sum(m_cls, axis=0, keepdims=True)
    m_bf = m_cls.astype(jnp.bfloat16)

    eg = jnp.exp(_GAMMA * preds)
    u_bf = (m_cls / eg).astype(jnp.bfloat16)

    a_bf = a_ref[...].astype(jnp.bfloat16)
    aself_bf = aself_ref[...].astype(jnp.bfloat16)
    inter = jax.lax.dot_general(a_bf, a_bf, (((1,), (0,)), ((), ())),
                                preferred_element_type=jnp.float32)
    cross = jax.lax.dot_general(a_bf, aself_bf, (((1,), (0,)), ((), ())),
                                preferred_element_type=jnp.float32)
    wt = jax.lax.dot_general(a_bf, m_bf, (((1,), (0,)), ((), ())),
                             preferred_element_type=jnp.float32)

    ones_row = jnp.ones((1, _N), dtype=jnp.bfloat16)
    deg_row = jax.lax.dot_general(ones_row, a_bf, (((1,), (0,)), ((), ())),
                                  preferred_element_type=jnp.float32)
    base_row = 1.0 + _SIG1 * deg_row

    t_parts = []
    for c in range(_NCHUNK):
        r0 = c * _CHUNK
        ic = jax.lax.slice(inter, (r0, 0), (r0 + _CHUNK, _N))
        cc = jax.lax.slice(cross, (r0, 0), (r0 + _CHUNK, _N))
        num = base_row - _SIG1 * cc
        rden = pl.reciprocal(1.0 + _SIG1 * ic, approx=True, full_range=False)
        v = pl.reciprocal(1.0 + jnp.exp(num * rden), approx=True,
                          full_range=False)
        t_parts.append(
            jax.lax.dot_general(v.astype(jnp.bfloat16), u_bf,
                                (((1,), (0,)), ((), ())),
                                preferred_element_type=jnp.float32))
    t = jnp.concatenate(t_parts, axis=0)

    p_t = jax.lax.dot_general(m_cls, t * eg, (((0,), (0,)), ((), ())),
                              preferred_element_type=jnp.float32)

    zt = ncnt - wt - m_cls * (1.0 - diag_col)
    g_inter = jax.lax.dot_general(wt, wt, (((0,), (0,)), ((), ())),
                                  preferred_element_type=jnp.float32)
    g_sub_t = jax.lax.dot_general(zt, wt, (((0,), (0,)), ((), ())),
                                  preferred_element_type=jnp.float32)

    denom = jnp.reshape(ncnt, (_C, 1)) * ncnt
    recip = jnp.where(denom > 0.0, 1.0 / jnp.where(denom > 0.0, denom, 1.0), 0.0)
    ii = jax.lax.broadcasted_iota(jnp.int32, (_C, _C), 0)
    jj = jax.lax.broadcasted_iota(jnp.int32, (_C, _C), 1)
    keep = jnp.logical_and(jnp.logical_and(g_sub_t > 0.0, g_inter > 0.0),
                           ii != jj)
    pair_loss = jnp.sum(jnp.where(keep, p_t * recip, 0.0))

    out_ref[...] = jnp.reshape(ce + _PER * pair_loss, (1, 1))


def kernel(preds, labels, mask, w_values_dict, adj_matrix):
    del w_values_dict
    adj_b = adj_matrix.astype(bool)
    aself_b = jnp.logical_or(adj_b, jnp.eye(_N, dtype=bool))
    diag_col = jnp.diagonal(adj_b).astype(jnp.float32).reshape(_N, 1)
    lab = labels.astype(jnp.int32).reshape(_N, 1)
    maskf = mask.astype(jnp.float32).reshape(_N, 1)
    out = pl.pallas_call(
        _loss_body,
        out_shape=jax.ShapeDtypeStruct((1, 1), jnp.float32),
    )(preds.astype(jnp.float32), lab, maskf, adj_b, aself_b, diag_col)
    return out[0, 0]


# speedup vs baseline: 1.0692x; 1.0128x over previous
import jax
import jax.numpy as jnp
import numpy as np
from jax.experimental import pallas as pl

_N = 1024
_C = 8
_GAMMA = 1.0
_PER = 0.001
_SIG1 = float(1.0 / (1.0 + np.exp(-1.0)))


def _loss_body(preds_ref, lab_ref, maskf_ref, a_ref, aself_ref, diag_ref,
               out_ref):
    preds = preds_ref[...]
    labels = lab_ref[...]
    maskf = maskf_ref[...]
    diag_col = diag_ref[...]

    mx = jnp.max(preds, axis=1, keepdims=True)
    lse = jnp.log(jnp.sum(jnp.exp(preds - mx), axis=1, keepdims=True)) + mx
    logp = preds - lse
    cls_iota = jax.lax.broadcasted_iota(jnp.int32, (_N, _C), 1)
    lab_oh = (cls_iota == labels).astype(jnp.float32)
    ce = -jnp.sum(logp * lab_oh) * (1.0 / _N)

    m_cls = lab_oh * maskf
    ncnt = jnp.sum(m_cls, axis=0, keepdims=True)
    m_bf = m_cls.astype(jnp.bfloat16)

    eg = jnp.exp(_GAMMA * preds)
    u_bf = (m_cls / eg).astype(jnp.bfloat16)

    a_bf = a_ref[...].astype(jnp.bfloat16)
    aself_bf = aself_ref[...].astype(jnp.bfloat16)
    inter = jax.lax.dot_general(a_bf, a_bf, (((1,), (0,)), ((), ())),
                                preferred_element_type=jnp.float32
                                ).astype(jnp.bfloat16)
    cross = jax.lax.dot_general(a_bf, aself_bf, (((1,), (0,)), ((), ())),
                                preferred_element_type=jnp.float32
                                ).astype(jnp.bfloat16)
    wt = jax.lax.dot_general(a_bf, m_bf, (((1,), (0,)), ((), ())),
                             preferred_element_type=jnp.float32)

    ones_row = jnp.ones((1, _N), dtype=jnp.bfloat16)
    deg_row = jax.lax.dot_general(ones_row, a_bf, (((1,), (0,)), ((), ())),
                                  preferred_element_type=jnp.float32
                                  ).astype(jnp.bfloat16)

    s = jnp.bfloat16(_SIG1)
    one = jnp.bfloat16(1.0)
    base_row = one + s * deg_row
    num = base_row - s * cross
    den = one + s * inter
    v_bf = one / (one + jnp.exp(num / den))

    t = jax.lax.dot_general(v_bf, u_bf, (((1,), (0,)), ((), ())),
                            preferred_element_type=jnp.float32)
    p_t = jax.lax.dot_general(m_cls, t * eg, (((0,), (0,)), ((), ())),
                              preferred_element_type=jnp.float32)

    zt = ncnt - wt - m_cls * (1.0 - diag_col)
    g_inter = jax.lax.dot_general(wt, wt, (((0,), (0,)), ((), ())),
                                  preferred_element_type=jnp.float32)
    g_sub_t = jax.lax.dot_general(zt, wt, (((0,), (0,)), ((), ())),
                                  preferred_element_type=jnp.float32)

    denom = jnp.reshape(ncnt, (_C, 1)) * ncnt
    recip = jnp.where(denom > 0.0, 1.0 / jnp.where(denom > 0.0, denom, 1.0), 0.0)
    ii = jax.lax.broadcasted_iota(jnp.int32, (_C, _C), 0)
    jj = jax.lax.broadcasted_iota(jnp.int32, (_C, _C), 1)
    keep = jnp.logical_and(jnp.logical_and(g_sub_t > 0.0, g_inter > 0.0),
                           ii != jj)
    pair_loss = jnp.sum(jnp.where(keep, p_t * recip, 0.0))

    out_ref[...] = jnp.reshape(ce + _PER * pair_loss, (1, 1))


def kernel(preds, labels, mask, w_values_dict, adj_matrix):
    del w_values_dict
    adj_b = adj_matrix.astype(bool)
    aself_b = jnp.logical_or(adj_b, jnp.eye(_N, dtype=bool))
    diag_col = jnp.diagonal(adj_b).astype(jnp.float32).reshape(_N, 1)
    lab = labels.astype(jnp.int32).reshape(_N, 1)
    maskf = mask.astype(jnp.float32).reshape(_N, 1)
    out = pl.pallas_call(
        _loss_body,
        out_shape=jax.ShapeDtypeStruct((1, 1), jnp.float32),
    )(preds.astype(jnp.float32), lab, maskf, adj_b, aself_b, diag_col)
    return out[0, 0]


# speedup vs baseline: 1.9855x; 1.8570x over previous
import jax
import jax.numpy as jnp
import numpy as np
from jax.experimental import pallas as pl

_N = 1024
_C = 8
_GAMMA = 1.0
_PER = 0.001
_SIG1 = float(1.0 / (1.0 + np.exp(-1.0)))


def _loss_body(preds_ref, lab_ref, mask_ref, a_ref, aself_ref, out_ref):
    preds = preds_ref[...]
    lab8 = lab_ref[...]
    mask8 = mask_ref[...]

    mx = jnp.max(preds, axis=1, keepdims=True)
    lse = jnp.log(jnp.sum(jnp.exp(preds - mx), axis=1, keepdims=True)) + mx
    logp = preds - lse
    cls_iota = jax.lax.broadcasted_iota(jnp.int32, (_N, _C), 1)
    lab_oh = (cls_iota == lab8).astype(jnp.float32)
    ce = -jnp.sum(logp * lab_oh) * (1.0 / _N)

    m_cls = lab_oh * mask8
    ncnt = jnp.sum(m_cls, axis=0, keepdims=True)
    m_bf = m_cls.astype(jnp.bfloat16)

    eg = jnp.exp(_GAMMA * preds)
    u_bf = (m_cls / eg).astype(jnp.bfloat16)

    a_bf = a_ref[...].astype(jnp.bfloat16)
    aself_bf = aself_ref[...].astype(jnp.bfloat16)
    inter = jax.lax.dot_general(a_bf, a_bf, (((1,), (0,)), ((), ())),
                                preferred_element_type=jnp.float32
                                ).astype(jnp.bfloat16)
    cross = jax.lax.dot_general(a_bf, aself_bf, (((1,), (0,)), ((), ())),
                                preferred_element_type=jnp.float32
                                ).astype(jnp.bfloat16)
    wt = jax.lax.dot_general(a_bf, m_bf, (((1,), (0,)), ((), ())),
                             preferred_element_type=jnp.float32)
    wself = jax.lax.dot_general(aself_bf, m_bf, (((1,), (0,)), ((), ())),
                                preferred_element_type=jnp.float32)

    ones_row = jnp.ones((1, _N), dtype=jnp.bfloat16)
    deg_row = jax.lax.dot_general(ones_row, a_bf, (((1,), (0,)), ((), ())),
                                  preferred_element_type=jnp.float32
                                  ).astype(jnp.bfloat16)

    s = jnp.bfloat16(_SIG1)
    one = jnp.bfloat16(1.0)
    base_row = one + s * deg_row
    num = base_row - s * cross
    den = one + s * inter
    v_bf = one / (one + jnp.exp(num / den))

    t = jax.lax.dot_general(v_bf, u_bf, (((1,), (0,)), ((), ())),
                            preferred_element_type=jnp.float32)
    p_t = jax.lax.dot_general(m_cls, t * eg, (((0,), (0,)), ((), ())),
                              preferred_element_type=jnp.float32)

    zt = ncnt - wself
    g_inter = jax.lax.dot_general(wt, wt, (((0,), (0,)), ((), ())),
                                  preferred_element_type=jnp.float32)
    g_sub_t = jax.lax.dot_general(zt, wt, (((0,), (0,)), ((), ())),
                                  preferred_element_type=jnp.float32)

    denom = jnp.reshape(ncnt, (_C, 1)) * ncnt
    recip = jnp.where(denom > 0.0, 1.0 / jnp.where(denom > 0.0, denom, 1.0), 0.0)
    ii = jax.lax.broadcasted_iota(jnp.int32, (_C, _C), 0)
    jj = jax.lax.broadcasted_iota(jnp.int32, (_C, _C), 1)
    keep = jnp.logical_and(jnp.logical_and(g_sub_t > 0.0, g_inter > 0.0),
                           ii != jj)
    pair_loss = jnp.sum(jnp.where(keep, p_t * recip, 0.0))

    out_ref[...] = jnp.reshape(ce + _PER * pair_loss, (1, 1))


def kernel(preds, labels, mask, w_values_dict, adj_matrix):
    del w_values_dict
    adj_b = adj_matrix.astype(bool)
    aself_b = jnp.logical_or(adj_b, jnp.eye(_N, dtype=bool))
    lab8 = jnp.broadcast_to(labels.astype(jnp.int32)[:, None], (_N, _C))
    mask8 = jnp.broadcast_to(mask.astype(jnp.float32)[:, None], (_N, _C))
    out = pl.pallas_call(
        _loss_body,
        out_shape=jax.ShapeDtypeStruct((1, 1), jnp.float32),
    )(preds.astype(jnp.float32), lab8, mask8, adj_b, aself_b)
    return out[0, 0]
